# baseline (speedup 1.0000x reference)
import functools

import jax
import jax.numpy as jnp
from jax import lax
from jax.experimental import pallas as pl
from jax.experimental.pallas import tpu as pltpu
from jax.experimental.pallas import tpu_sc as plsc

_EPS = 0.1
_ROW_BLK = 512
_VOCAB_BLK = 4096


def _gather_body(num_rows, num_classes, bpw, nc,
                 pred_hbm, tgt_hbm, out_hbm, idx_v, chunks_v, vals_v, sem):
    wid = lax.axis_index("s") * nc + lax.axis_index("c")
    base = wid * bpw
    pltpu.sync_copy(tgt_hbm.at[pl.ds(base, bpw)], idx_v)

    copies = []
    for g in range(bpw // 16):
        tv = idx_v[pl.ds(g * 16, 16)]
        tqv = lax.shift_right_logical(tv, 7)
        for l in range(16):
            i = g * 16 + l
            r0 = ((base + i) // 8) * 8
            c0 = tqv[l] * 128
            copies.append(pltpu.async_copy(
                pred_hbm.at[pl.ds(r0, 8), pl.ds(c0, 128)],
                chunks_v.at[i], sem))
    for cp in copies:
        cp.wait()

    for g in range(bpw // 16):
        tv = idx_v[pl.ds(g * 16, 16)]
        c0v = lax.shift_left(lax.shift_right_logical(tv, 7), 7)
        lanev = tv - c0v
        rowv = lax.iota(jnp.int32, 16) + g * 16
        subv = (rowv + base) % 8
        vals_v[pl.ds(g * 16, 16)] = plsc.load_gather(
            chunks_v, [rowv, subv, lanev])
    pltpu.sync_copy(vals_v, out_hbm.at[pl.ds(base, bpw)])


def _gather_target(pred, target):
    n, k = pred.shape
    info = plsc.get_sparse_core_info()
    nc, ns = info.num_cores, info.num_subcores
    nw = nc * ns
    bpw = n // nw
    mesh = plsc.VectorSubcoreMesh(core_axis_name="c", subcore_axis_name="s")
    body = functools.partial(_gather_body, n, k, bpw, nc)
    return pl.kernel(
        body,
        mesh=mesh,
        compiler_params=pltpu.CompilerParams(needs_layout_passes=False),
        out_type=jax.ShapeDtypeStruct((n,), jnp.float32),
        scratch_types=[
            pltpu.VMEM((bpw,), jnp.int32),
            pltpu.VMEM((bpw, 8, 128), jnp.float32),
            pltpu.VMEM((bpw,), jnp.float32),
            pltpu.SemaphoreType.DMA,
        ],
    )(pred, target)


def _loss_body(nv, num_classes, num_rows, x_ref, out_ref,
               m_ref, s_ref, su_ref):
    j = pl.program_id(1)
    rb = x_ref.shape[0]
    vb = x_ref.shape[1]

    @pl.when(j == 0)
    def _init_acc():
        m_ref[...] = jnp.full((rb, 1), -jnp.inf, jnp.float32)
        s_ref[...] = jnp.zeros((rb, 1), jnp.float32)
        su_ref[...] = jnp.zeros((rb, 1), jnp.float32)

    def update(x, xm):
        bm = jnp.max(xm, axis=1, keepdims=True)
        m_new = jnp.maximum(m_ref[...], bm)
        alpha = jnp.exp(m_ref[...] - m_new)
        s_ref[...] = s_ref[...] * alpha + jnp.sum(
            jnp.exp(xm - m_new), axis=1, keepdims=True)
        su_ref[...] += jnp.sum(x, axis=1, keepdims=True)
        m_ref[...] = m_new

    @pl.when(j < nv - 1)
    def _full_block():
        x = x_ref[...]
        update(x, x)

    @pl.when(j == nv - 1)
    def _last_block():
        x = x_ref[...]
        cols = j * vb + lax.broadcasted_iota(jnp.int32, (1, vb), 1)
        valid = cols < num_classes
        xm = jnp.where(valid, x, -jnp.inf)
        x0 = jnp.where(valid, x, 0.0)
        update(x0, xm)
        a = _EPS / (num_classes - 1)
        c = 1.0 - _EPS - a
        lse = m_ref[...] + jnp.log(s_ref[...])
        base_rows = (a * num_classes + c) * lse - a * su_ref[...]
        out_ref[...] = jnp.sum(base_rows).reshape(1, 1, 1)


def _combine_body(num_classes, num_rows, base_ref, pt_ref, out_ref):
    a = _EPS / (num_classes - 1)
    c = 1.0 - _EPS - a
    total = jnp.sum(base_ref[...]) - c * jnp.sum(pt_ref[...])
    out_ref[...] = total.reshape(1, 1) / num_rows


def kernel(pred, target):
    n, k = pred.shape
    rb = _ROW_BLK
    vb = _VOCAB_BLK
    nr = n // rb
    nv = -(-k // vb)

    tgt = target.astype(jnp.int32)
    p_t = _gather_target(pred, tgt)

    body = functools.partial(_loss_body, nv, k, n)
    base = pl.pallas_call(
        body,
        grid=(nr, nv),
        in_specs=[
            pl.BlockSpec((rb, vb), lambda r, j: (r, j)),
        ],
        out_specs=pl.BlockSpec((1, 1, 1), lambda r, j: (r, 0, 0)),
        out_shape=jax.ShapeDtypeStruct((nr, 1, 1), jnp.float32),
        scratch_shapes=[
            pltpu.VMEM((rb, 1), jnp.float32),
            pltpu.VMEM((rb, 1), jnp.float32),
            pltpu.VMEM((rb, 1), jnp.float32),
        ],
        compiler_params=pltpu.CompilerParams(
            dimension_semantics=("parallel", "arbitrary"),
            vmem_limit_bytes=100 * 1024 * 1024),
    )(pred)
    out = pl.pallas_call(
        functools.partial(_combine_body, k, n),
        in_specs=[
            pl.BlockSpec((nr, 1, 1), lambda: (0, 0, 0)),
            pl.BlockSpec((n, 1), lambda: (0, 0)),
        ],
        out_specs=pl.BlockSpec((1, 1), lambda: (0, 0)),
        out_shape=jax.ShapeDtypeStruct((1, 1), jnp.float32),
    )(base, p_t.reshape(n, 1))
    return out.reshape(())


# speedup vs baseline: 1.0259x; 1.0259x over previous
import functools

import jax
import jax.numpy as jnp
from jax import lax
from jax.experimental import pallas as pl
from jax.experimental.pallas import tpu as pltpu
from jax.experimental.pallas import tpu_sc as plsc

_EPS = 0.1
_ROW_BLK = 1024
_VOCAB_BLK = 3584


def _gather_body(num_rows, num_classes, bpw, nc,
                 pred_hbm, tgt_hbm, out_hbm, idx_v, chunks_v, vals_v, sem):
    wid = lax.axis_index("s") * nc + lax.axis_index("c")
    base = wid * bpw
    pltpu.sync_copy(tgt_hbm.at[pl.ds(base, bpw)], idx_v)

    copies = []
    for g in range(bpw // 16):
        tv = idx_v[pl.ds(g * 16, 16)]
        tqv = lax.shift_right_logical(tv, 7)
        for l in range(16):
            i = g * 16 + l
            r0 = ((base + i) // 8) * 8
            c0 = tqv[l] * 128
            copies.append(pltpu.async_copy(
                pred_hbm.at[pl.ds(r0, 8), pl.ds(c0, 128)],
                chunks_v.at[i], sem))
    for cp in copies:
        cp.wait()

    for g in range(bpw // 16):
        tv = idx_v[pl.ds(g * 16, 16)]
        c0v = lax.shift_left(lax.shift_right_logical(tv, 7), 7)
        lanev = tv - c0v
        rowv = lax.iota(jnp.int32, 16) + g * 16
        subv = (rowv + base) % 8
        vals_v[pl.ds(g * 16, 16)] = plsc.load_gather(
            chunks_v, [rowv, subv, lanev])
    pltpu.sync_copy(vals_v, out_hbm.at[pl.ds(base, bpw)])


def _gather_target(pred, target):
    n, k = pred.shape
    info = plsc.get_sparse_core_info()
    nc, ns = info.num_cores, info.num_subcores
    nw = nc * ns
    bpw = n // nw
    mesh = plsc.VectorSubcoreMesh(core_axis_name="c", subcore_axis_name="s")
    body = functools.partial(_gather_body, n, k, bpw, nc)
    return pl.kernel(
        body,
        mesh=mesh,
        compiler_params=pltpu.CompilerParams(needs_layout_passes=False),
        out_type=jax.ShapeDtypeStruct((n,), jnp.float32),
        scratch_types=[
            pltpu.VMEM((bpw,), jnp.int32),
            pltpu.VMEM((bpw, 8, 128), jnp.float32),
            pltpu.VMEM((bpw,), jnp.float32),
            pltpu.SemaphoreType.DMA,
        ],
    )(pred, target)


def _loss_body(nv, num_classes, num_rows, x_ref, out_ref,
               m_ref, s_ref, su_ref):
    j = pl.program_id(1)
    rb = x_ref.shape[0]
    vb = x_ref.shape[1]

    @pl.when(j == 0)
    def _init_acc():
        m_ref[...] = jnp.full((rb, 1), -jnp.inf, jnp.float32)
        s_ref[...] = jnp.zeros((rb, 1), jnp.float32)
        su_ref[...] = jnp.zeros((rb, 1), jnp.float32)

    def update(x, xm):
        bm = jnp.max(xm, axis=1, keepdims=True)
        m_new = jnp.maximum(m_ref[...], bm)
        alpha = jnp.exp(m_ref[...] - m_new)
        s_ref[...] = s_ref[...] * alpha + jnp.sum(
            jnp.exp(xm - m_new), axis=1, keepdims=True)
        su_ref[...] += jnp.sum(x, axis=1, keepdims=True)
        m_ref[...] = m_new

    @pl.when(j < nv - 1)
    def _full_block():
        x = x_ref[...]
        update(x, x)

    @pl.when(j == nv - 1)
    def _last_block():
        x = x_ref[...]
        cols = j * vb + lax.broadcasted_iota(jnp.int32, (1, vb), 1)
        valid = cols < num_classes
        xm = jnp.where(valid, x, -jnp.inf)
        x0 = jnp.where(valid, x, 0.0)
        update(x0, xm)
        a = _EPS / (num_classes - 1)
        c = 1.0 - _EPS - a
        lse = m_ref[...] + jnp.log(s_ref[...])
        base_rows = (a * num_classes + c) * lse - a * su_ref[...]
        out_ref[...] = jnp.sum(base_rows).reshape(1, 1, 1)


def _combine_body(num_classes, num_rows, base_ref, pt_ref, out_ref):
    a = _EPS / (num_classes - 1)
    c = 1.0 - _EPS - a
    total = jnp.sum(base_ref[...]) - c * jnp.sum(pt_ref[...])
    out_ref[...] = total.reshape(1, 1) / num_rows


def kernel(pred, target):
    n, k = pred.shape
    rb = _ROW_BLK
    vb = _VOCAB_BLK
    nr = n // rb
    nv = -(-k // vb)

    tgt = target.astype(jnp.int32)
    p_t = _gather_target(pred, tgt)

    body = functools.partial(_loss_body, nv, k, n)
    base = pl.pallas_call(
        body,
        grid=(nr, nv),
        in_specs=[
            pl.BlockSpec((rb, vb), lambda r, j: (r, j)),
        ],
        out_specs=pl.BlockSpec((1, 1, 1), lambda r, j: (r, 0, 0)),
        out_shape=jax.ShapeDtypeStruct((nr, 1, 1), jnp.float32),
        scratch_shapes=[
            pltpu.VMEM((rb, 1), jnp.float32),
            pltpu.VMEM((rb, 1), jnp.float32),
            pltpu.VMEM((rb, 1), jnp.float32),
        ],
        compiler_params=pltpu.CompilerParams(
            dimension_semantics=("parallel", "arbitrary"),
            vmem_limit_bytes=100 * 1024 * 1024),
    )(pred)
    out = pl.pallas_call(
        functools.partial(_combine_body, k, n),
        in_specs=[
            pl.BlockSpec((nr, 1, 1), lambda: (0, 0, 0)),
            pl.BlockSpec((n, 1), lambda: (0, 0)),
        ],
        out_specs=pl.BlockSpec((1, 1), lambda: (0, 0)),
        out_shape=jax.ShapeDtypeStruct((1, 1), jnp.float32),
    )(base, p_t.reshape(n, 1))
    return out.reshape(())
